# baseline (speedup 1.0000x reference)
import functools

import jax
import jax.numpy as jnp
from jax import lax
from jax.experimental import pallas as pl
from jax.experimental.pallas import tpu as pltpu
from jax.experimental.pallas import tpu_sc as plsc

_B = 16384
_D = 64
_NC = 2
_NS = 16
_NW = _NC * _NS
_BPW = _B // _NW
_AVG = 3.5
_L = 16
_DH = _D // 2
_NG = _BPW // _L


def _body(xt_hbm, user_hbm, item_hbm, wb_hbm, ubias_hbm, ibias_hbm,
          out_hbm, uidx_v, iidx_v, ub_v, ib_v, x_v, wb_v, wsp_v, out_v,
          gsem, xsem0, xsem1):
    wid = lax.axis_index("s") * _NC + lax.axis_index("c")
    base = wid * _BPW

    x0 = pltpu.async_copy(xt_hbm.at[pl.ds(0, _DH), pl.ds(base, _BPW)],
                          x_v.at[pl.ds(0, _DH)], xsem0)

    ui = pltpu.async_copy(user_hbm.at[pl.ds(base, _BPW)], uidx_v, gsem)
    ii = pltpu.async_copy(item_hbm.at[pl.ds(base, _BPW)], iidx_v, gsem)
    ui.wait()
    ii.wait()
    ug = pltpu.async_copy(ubias_hbm.at[uidx_v], ub_v, gsem)
    ig = pltpu.async_copy(ibias_hbm.at[iidx_v], ib_v, gsem)

    x1 = pltpu.async_copy(xt_hbm.at[pl.ds(_DH, _DH), pl.ds(base, _BPW)],
                          x_v.at[pl.ds(_DH, _DH)], xsem1)

    pltpu.sync_copy(wb_hbm, wb_v)

    wv = [wb_v[pl.ds(k * _L, _L)] for k in range(_D // _L)]
    for j in range(_D):
        wsp_v[pl.ds(j * _L, _L)] = jnp.broadcast_to(wv[j // _L][j % _L],
                                                    (_L,))
    bconst = wb_v[pl.ds(_D, _L)]

    def feat(j, accs):
        wj = wsp_v[pl.ds(j * _L, _L)]
        return tuple(accs[g] + x_v[j, pl.ds(g * _L, _L)] * wj
                     for g in range(_NG))

    init = tuple(bconst for _ in range(_NG))
    x0.wait()
    x1.wait()
    accs = lax.fori_loop(0, _D, feat, init)

    ug.wait()
    ig.wait()

    for g in range(_NG):
        c = g * _L
        out_v[pl.ds(c, _L)] = accs[g] + (ub_v[pl.ds(c, _L)]
                                         + ib_v[pl.ds(c, _L)])

    pltpu.sync_copy(out_v, out_hbm.at[pl.ds(base, _BPW)])


@jax.jit
def _run(xt, user, item, wb, ubias_flat, ibias_flat):
    mesh = plsc.VectorSubcoreMesh(core_axis_name="c", subcore_axis_name="s")
    f = functools.partial(
        pl.kernel,
        out_type=jax.ShapeDtypeStruct((_B,), jnp.float32),
        mesh=mesh,
        scratch_types=[
            pltpu.VMEM((_BPW,), jnp.int32),
            pltpu.VMEM((_BPW,), jnp.int32),
            pltpu.VMEM((_BPW,), jnp.float32),
            pltpu.VMEM((_BPW,), jnp.float32),
            pltpu.VMEM((_D, _BPW), jnp.float32),
            pltpu.VMEM((_D + _L,), jnp.float32),
            pltpu.VMEM((_D * _L,), jnp.float32),
            pltpu.VMEM((_BPW,), jnp.float32),
            pltpu.SemaphoreType.DMA,
            pltpu.SemaphoreType.DMA,
            pltpu.SemaphoreType.DMA,
        ],
    )(_body)
    return f(xt, user, item, wb, ubias_flat, ibias_flat)


def kernel(mlp_concat_emebd, user, item, W, b, user_bias, item_bias):
    wb = jnp.concatenate(
        [W.reshape(-1), jnp.broadcast_to(b.reshape(1) + _AVG, (_L,))])
    return _run(mlp_concat_emebd.T, user.astype(jnp.int32),
                item.astype(jnp.int32), wb,
                user_bias.reshape(-1), item_bias.reshape(-1))
